# baseline (speedup 1.0000x reference)
import functools

import jax
import jax.numpy as jnp
from jax import lax
from jax.experimental import pallas as pl
from jax.experimental.pallas import tpu as pltpu
from jax.experimental.pallas import tpu_sc as plsc

B = 16384
D = 64
K = 8
TAIL = 3.0
MIN_W = 0.001
MIN_H = 0.001
MIN_D = 0.001
CW = 2 * TAIL - K * MIN_W
CH = 2 * TAIL - K * MIN_H
LN2 = 0.6931471805599453
SQRTH = 1.4142135381698608
L1P = (0.010243828, -0.05326748, 0.13198966, -0.2239669,
       0.32751173, -0.49933395, 0.99997026, 2.2159765e-07)
PLOG = (-1.1514610310e-1, 1.1676998740e-1, -1.2420140846e-1,
        1.4249322787e-1, -1.6668057665e-1, 2.0000714765e-1,
        -2.4999993993e-1, 3.3333331174e-1)

NW = 32
B_SC = 4096
B_TC = B - B_SC
BT = 2048
DB = 8
TCW = 256
OFF = B_SC // BT
BW = B_SC // NW
NG = BW // 16


def _plog(v):
    bits = plsc.bitcast(v, jnp.int32)
    e = lax.shift_right_logical(bits, 23) - 127
    m = plsc.bitcast((bits & 0x007FFFFF) | 0x3F800000, jnp.float32)
    big = m > SQRTH
    m = jnp.where(big, m * 0.5, m)
    e = e + jnp.where(big, 1, 0)
    r = m - 1.0
    z = r * r
    p = jnp.float32(7.0376836292e-2)
    for c in PLOG:
        p = p * r + c
    return r + (r * z * p - 0.5 * z) + e.astype(jnp.float32) * LN2


def _softplus(u):
    t = jnp.exp(-jnp.abs(u))
    p = jnp.float32(L1P[0])
    for c in L1P[1:]:
        p = p * t + c
    return jnp.maximum(u, 0.0) + p


def _sc_body(x_hbm, uw_hbm, uh_hbm, ud_hbm, out_hbm, ld_hbm,
             xv, uwv, uhv, udv, outv, ldv,
             in_sem0, in_sem1, out_sem0, out_sem1):
    wid = lax.axis_index("s") * 2 + lax.axis_index("c")
    b0 = pl.multiple_of(wid * BW, BW)
    bsl = pl.ds(b0, BW)
    in_sems = (in_sem0, in_sem1)
    out_sems = (out_sem0, out_sem1)

    @plsc.parallel_loop(0, NG)
    def _zero(g):
        ldv[pl.ds(g * 16, 16)] = jnp.zeros((16,), jnp.float32)

    def in_copies(d, slot):
        sem = in_sems[slot]
        ssl = pl.ds(slot * BW, BW)
        return [
            pltpu.make_async_copy(x_hbm.at[d, bsl], xv.at[ssl], sem),
            pltpu.make_async_copy(uw_hbm.at[d, :, bsl], uwv.at[:, ssl], sem),
            pltpu.make_async_copy(uh_hbm.at[d, :, bsl], uhv.at[:, ssl], sem),
            pltpu.make_async_copy(ud_hbm.at[:, d, bsl], udv.at[:, ssl], sem),
        ]

    def issue_in(d, slot):
        for cp in in_copies(d, slot):
            cp.start()

    def wait_in(d, slot):
        for cp in in_copies(d, slot):
            cp.wait()

    issue_in(0, 0)
    issue_in(1, 1)

    def process(d, slot):
        wait_in(d, slot)

        @pl.when(d >= 2)
        def _():
            pltpu.make_async_copy(outv.at[pl.ds(slot * BW, BW)],
                                  out_hbm.at[d, bsl], out_sems[slot]).wait()

        @plsc.parallel_loop(0, NG)
        def _grp(g):
            o = g * 16
            sl = pl.ds(slot * BW + o, 16)
            x = xv[sl]
            tw = [jnp.exp(uwv[k, sl]) for k in range(K)]
            th = [jnp.exp(uhv[k, sl]) for k in range(K)]
            u = [udv[k, sl] for k in range(K + 1)]

            sw = tw[0]
            for k in range(1, K):
                sw = sw + tw[k]
            fw = CW / sw
            cwk = [tw[k] * fw for k in range(K)]
            kx = [jnp.full((16,), -TAIL, jnp.float32)]
            for k in range(K):
                kx.append(kx[k] + (MIN_W + cwk[k]))

            sh = th[0]
            for k in range(1, K):
                sh = sh + th[k]
            fh = CH / sh
            chk = [th[k] * fh for k in range(K)]
            ky = [jnp.full((16,), -TAIL, jnp.float32)]
            for k in range(K):
                ky.append(ky[k] + (MIN_H + chk[k]))

            m = [kx[k] <= x for k in range(1, K)]

            def ladder(vals, shift=0):
                r = vals[shift]
                for j in range(1, len(vals) - shift):
                    r = jnp.where(m[j - 1], vals[j + shift], r)
                return r

            x_k = ladder(kx[:K])
            y_k = ladder(ky[:K])
            w_b = MIN_W + ladder(cwk)
            h_b = MIN_H + ladder(chk)
            d_k = MIN_D + _softplus(ladder(u[:K]))
            d_k1 = MIN_D + _softplus(ladder(u, shift=1))

            inv_w = 1.0 / w_b
            xi = (x - x_k) * inv_w
            s_k = h_b * inv_w
            omx = 1.0 - xi
            ximx = xi * omx
            xi2 = xi * xi
            num = h_b * (s_k * xi2 + d_k * ximx)
            den = s_k + (d_k1 - s_k + d_k - s_k) * ximx
            r_den = 1.0 / den
            y = y_k + num * r_den
            num_g = (s_k * s_k) * (d_k1 * xi2 + (s_k + s_k) * ximx
                                   + d_k * omx * omx)
            dy_dx = num_g * r_den * r_den * inv_w
            log_det = _plog(jnp.abs(dy_dx))

            outside = (x <= -TAIL) | (x >= TAIL)
            outv[sl] = jnp.where(outside, x, y)
            lsl = pl.ds(o, 16)
            ldv[lsl] = ldv[lsl] + jnp.where(outside, 0.0, log_det)

        pltpu.async_copy(outv.at[pl.ds(slot * BW, BW)], out_hbm.at[d, bsl],
                         out_sems[slot])

        @pl.when(d + 2 < D)
        def _():
            issue_in(d + 2, slot)

    def col_pair(dp, _):
        process(dp * 2, 0)
        process(dp * 2 + 1, 1)
        return 0

    lax.fori_loop(0, D // 2, col_pair, 0)

    pltpu.make_async_copy(outv.at[pl.ds(0, BW)], out_hbm.at[D - 2, bsl],
                          out_sem0).wait()
    pltpu.make_async_copy(outv.at[pl.ds(BW, BW)], out_hbm.at[D - 1, bsl],
                          out_sem1).wait()
    pltpu.sync_copy(ldv, ld_hbm.at[bsl])


def _tc_body(xr, uwr, uhr, udr, outr, ldr):
    for c in range(1):
        csl = slice(None)
        x = xr[:, csl]
        tw = [jnp.exp(uwr[:, k, csl]) for k in range(K)]
        th = [jnp.exp(uhr[:, k, csl]) for k in range(K)]
        u = [udr[k, :, csl] for k in range(K + 1)]

        sw = tw[0]
        for k in range(1, K):
            sw = sw + tw[k]
        fw = CW / sw
        cwk = [tw[k] * fw for k in range(K)]
        kx = [jnp.full(x.shape, -TAIL, jnp.float32)]
        for k in range(K):
            kx.append(kx[k] + (MIN_W + cwk[k]))

        sh = th[0]
        for k in range(1, K):
            sh = sh + th[k]
        fh = CH / sh
        chk = [th[k] * fh for k in range(K)]
        ky = [jnp.full(x.shape, -TAIL, jnp.float32)]
        for k in range(K):
            ky.append(ky[k] + (MIN_H + chk[k]))

        m = [kx[k] <= x for k in range(1, K)]

        def ladder(vals, shift=0):
            r = vals[shift]
            for j in range(1, len(vals) - shift):
                r = jnp.where(m[j - 1], vals[j + shift], r)
            return r

        def softplus(v):
            return jnp.maximum(v, 0.0) + jnp.log1p(jnp.exp(-jnp.abs(v)))

        x_k = ladder(kx[:K])
        y_k = ladder(ky[:K])
        w_b = MIN_W + ladder(cwk)
        h_b = MIN_H + ladder(chk)
        d_k = MIN_D + softplus(ladder(u[:K]))
        d_k1 = MIN_D + softplus(ladder(u, shift=1))

        inv_w = 1.0 / w_b
        xi = (x - x_k) * inv_w
        s_k = h_b * inv_w
        omx = 1.0 - xi
        ximx = xi * omx
        xi2 = xi * xi
        num = h_b * (s_k * xi2 + d_k * ximx)
        den = s_k + (d_k1 - s_k + d_k - s_k) * ximx
        r_den = 1.0 / den
        y = y_k + num * r_den
        num_g = (s_k * s_k) * (d_k1 * xi2 + (s_k + s_k) * ximx
                               + d_k * omx * omx)
        dy_dx = num_g * r_den * r_den * inv_w
        log_det = jnp.log(jnp.abs(dy_dx))

        outside = (x <= -TAIL) | (x >= TAIL)
        outr[:, csl] = jnp.where(outside, x, y)
        ldr[...] = jnp.sum(jnp.where(outside, 0.0, log_det), axis=0)


@jax.jit
def _run(x, uw, uh, ud):
    mesh = plsc.VectorSubcoreMesh(core_axis_name="c", subcore_axis_name="s")
    f = pl.kernel(
        _sc_body,
        mesh=mesh,
        compiler_params=pltpu.CompilerParams(needs_layout_passes=False),
        out_type=(
            jax.ShapeDtypeStruct((D, B_SC), jnp.float32),
            jax.ShapeDtypeStruct((B_SC,), jnp.float32),
        ),
        scratch_types=[
            pltpu.VMEM((2 * BW,), jnp.float32),
            pltpu.VMEM((K, 2 * BW), jnp.float32),
            pltpu.VMEM((K, 2 * BW), jnp.float32),
            pltpu.VMEM((K + 1, 2 * BW), jnp.float32),
            pltpu.VMEM((2 * BW,), jnp.float32),
            pltpu.VMEM((BW,), jnp.float32),
            pltpu.SemaphoreType.DMA,
            pltpu.SemaphoreType.DMA,
            pltpu.SemaphoreType.DMA,
            pltpu.SemaphoreType.DMA,
        ],
    )
    x_t = x.T
    uw_t = uw.transpose(1, 2, 0)
    uh_t = uh.transpose(1, 2, 0)
    ud_t = ud.transpose(2, 1, 0)
    out_sc, ld_sc = f(x_t, uw_t, uh_t, ud_t)

    out_tc, ld_tc = pl.pallas_call(
        _tc_body,
        grid=(B_TC // BT,),
        in_specs=[
            pl.BlockSpec((D, BT), lambda i: (0, OFF + i)),
            pl.BlockSpec((D, K, BT), lambda i: (0, 0, OFF + i)),
            pl.BlockSpec((D, K, BT), lambda i: (0, 0, OFF + i)),
            pl.BlockSpec((K + 1, D, BT), lambda i: (0, 0, OFF + i)),
        ],
        out_specs=[
            pl.BlockSpec((D, BT), lambda i: (0, i)),
            pl.BlockSpec((BT,), lambda i: (i,)),
        ],
        out_shape=[
            jax.ShapeDtypeStruct((D, B_TC), jnp.float32),
            jax.ShapeDtypeStruct((B_TC,), jnp.float32),
        ],
    )(x_t, uw_t, uh_t, ud_t)

    out_t = jnp.concatenate([out_sc, out_tc], axis=1)
    ld = jnp.concatenate([ld_sc, ld_tc])
    return out_t.T, ld


def kernel(x, unnormalized_widths, unnormalized_heights, unnormalized_derivatives):
    return _run(x, unnormalized_widths, unnormalized_heights,
                unnormalized_derivatives)


# speedup vs baseline: 1.0030x; 1.0030x over previous
import jax
import jax.numpy as jnp
from jax import lax
from jax.experimental import pallas as pl
from jax.experimental.pallas import tpu as pltpu
from jax.experimental.pallas import tpu_sc as plsc

B = 16384
D = 64
K = 8
TAIL = 3.0
MIN_W = 0.001
MIN_H = 0.001
MIN_D = 0.001
CW = 2 * TAIL - K * MIN_W
CH = 2 * TAIL - K * MIN_H
LN2 = 0.6931471805599453
SQRTH = 1.4142135381698608
L1P = (0.010243828, -0.05326748, 0.13198966, -0.2239669,
       0.32751173, -0.49933395, 0.99997026, 2.2159765e-07)
PLOG = (-1.1514610310e-1, 1.1676998740e-1, -1.2420140846e-1,
        1.4249322787e-1, -1.6668057665e-1, 2.0000714765e-1,
        -2.4999993993e-1, 3.3333331174e-1)

NW = 32
B_SC = 4096
B_TC = B - B_SC
BT = 1024
OFF = B_SC // BT
BW = B_SC // NW
NG = BW // 16


def _plog(v):
    bits = plsc.bitcast(v, jnp.int32)
    e = lax.shift_right_logical(bits, 23) - 127
    m = plsc.bitcast((bits & 0x007FFFFF) | 0x3F800000, jnp.float32)
    big = m > SQRTH
    m = jnp.where(big, m * 0.5, m)
    e = e + jnp.where(big, 1, 0)
    r = m - 1.0
    z = r * r
    p = jnp.float32(7.0376836292e-2)
    for c in PLOG:
        p = p * r + c
    return r + (r * z * p - 0.5 * z) + e.astype(jnp.float32) * LN2


def _softplus(u):
    t = jnp.exp(-jnp.abs(u))
    p = jnp.float32(L1P[0])
    for c in L1P[1:]:
        p = p * t + c
    return jnp.maximum(u, 0.0) + p


def _sc_body(x_hbm, uw_hbm, uh_hbm, ud_hbm, out_hbm, ld_hbm,
             xv, uwv, uhv, udv, outv, ldv,
             in_sem0, in_sem1, out_sem0, out_sem1):
    wid = lax.axis_index("s") * 2 + lax.axis_index("c")
    b0 = pl.multiple_of(wid * BW, BW)
    bsl = pl.ds(b0, BW)
    in_sems = (in_sem0, in_sem1)
    out_sems = (out_sem0, out_sem1)

    @plsc.parallel_loop(0, NG)
    def _zero(g):
        ldv[pl.ds(g * 16, 16)] = jnp.zeros((16,), jnp.float32)

    def in_copies(d, slot):
        sem = in_sems[slot]
        ssl = pl.ds(slot * BW, BW)
        return [
            pltpu.make_async_copy(x_hbm.at[d, bsl], xv.at[ssl], sem),
            pltpu.make_async_copy(uw_hbm.at[d, :, bsl], uwv.at[:, ssl], sem),
            pltpu.make_async_copy(uh_hbm.at[d, :, bsl], uhv.at[:, ssl], sem),
            pltpu.make_async_copy(ud_hbm.at[:, d, bsl], udv.at[:, ssl], sem),
        ]

    def issue_in(d, slot):
        for cp in in_copies(d, slot):
            cp.start()

    def wait_in(d, slot):
        for cp in in_copies(d, slot):
            cp.wait()

    issue_in(0, 0)
    issue_in(1, 1)

    def process(d, slot):
        wait_in(d, slot)

        @pl.when(d >= 2)
        def _():
            pltpu.make_async_copy(outv.at[pl.ds(slot * BW, BW)],
                                  out_hbm.at[d, bsl], out_sems[slot]).wait()

        @plsc.parallel_loop(0, NG)
        def _grp(g):
            o = g * 16
            sl = pl.ds(slot * BW + o, 16)
            x = xv[sl]
            tw = [jnp.exp(uwv[k, sl]) for k in range(K)]
            th = [jnp.exp(uhv[k, sl]) for k in range(K)]
            u = [udv[k, sl] for k in range(K + 1)]

            sw = tw[0]
            for k in range(1, K):
                sw = sw + tw[k]
            fw = CW / sw
            cwk = [tw[k] * fw for k in range(K)]
            kx = [jnp.full((16,), -TAIL, jnp.float32)]
            for k in range(K):
                kx.append(kx[k] + (MIN_W + cwk[k]))

            sh = th[0]
            for k in range(1, K):
                sh = sh + th[k]
            fh = CH / sh
            chk = [th[k] * fh for k in range(K)]
            ky = [jnp.full((16,), -TAIL, jnp.float32)]
            for k in range(K):
                ky.append(ky[k] + (MIN_H + chk[k]))

            m = [kx[k] <= x for k in range(1, K)]

            def ladder(vals, shift=0):
                r = vals[shift]
                for j in range(1, len(vals) - shift):
                    r = jnp.where(m[j - 1], vals[j + shift], r)
                return r

            x_k = ladder(kx[:K])
            y_k = ladder(ky[:K])
            w_b = MIN_W + ladder(cwk)
            h_b = MIN_H + ladder(chk)
            d_k = MIN_D + _softplus(ladder(u[:K]))
            d_k1 = MIN_D + _softplus(ladder(u, shift=1))

            inv_w = 1.0 / w_b
            xi = (x - x_k) * inv_w
            s_k = h_b * inv_w
            omx = 1.0 - xi
            ximx = xi * omx
            xi2 = xi * xi
            num = h_b * (s_k * xi2 + d_k * ximx)
            den = s_k + (d_k1 - s_k + d_k - s_k) * ximx
            r_den = 1.0 / den
            y = y_k + num * r_den
            num_g = (s_k * s_k) * (d_k1 * xi2 + (s_k + s_k) * ximx
                                   + d_k * omx * omx)
            dy_dx = num_g * r_den * r_den * inv_w
            log_det = _plog(jnp.abs(dy_dx))

            outside = (x <= -TAIL) | (x >= TAIL)
            outv[sl] = jnp.where(outside, x, y)
            lsl = pl.ds(o, 16)
            ldv[lsl] = ldv[lsl] + jnp.where(outside, 0.0, log_det)

        pltpu.async_copy(outv.at[pl.ds(slot * BW, BW)], out_hbm.at[d, bsl],
                         out_sems[slot])

        @pl.when(d + 2 < D)
        def _():
            issue_in(d + 2, slot)

    def col_pair(dp, _):
        process(dp * 2, 0)
        process(dp * 2 + 1, 1)
        return 0

    lax.fori_loop(0, D // 2, col_pair, 0)

    pltpu.make_async_copy(outv.at[pl.ds(0, BW)], out_hbm.at[D - 2, bsl],
                          out_sem0).wait()
    pltpu.make_async_copy(outv.at[pl.ds(BW, BW)], out_hbm.at[D - 1, bsl],
                          out_sem1).wait()
    pltpu.sync_copy(ldv, ld_hbm.at[bsl])


def _tc_body(xr, uwr, uhr, udr, outr, ldr):
    x = xr[...]
    tw = [jnp.exp(uwr[:, k, :]) for k in range(K)]
    th = [jnp.exp(uhr[:, k, :]) for k in range(K)]
    u = [udr[k, :, :] for k in range(K + 1)]

    sw = tw[0]
    for k in range(1, K):
        sw = sw + tw[k]
    fw = CW / sw
    cwk = [tw[k] * fw for k in range(K)]
    kx = [jnp.full(x.shape, -TAIL, jnp.float32)]
    for k in range(K):
        kx.append(kx[k] + (MIN_W + cwk[k]))

    sh = th[0]
    for k in range(1, K):
        sh = sh + th[k]
    fh = CH / sh
    chk = [th[k] * fh for k in range(K)]
    ky = [jnp.full(x.shape, -TAIL, jnp.float32)]
    for k in range(K):
        ky.append(ky[k] + (MIN_H + chk[k]))

    m = [kx[k] <= x for k in range(1, K)]

    def ladder(vals, shift=0):
        r = vals[shift]
        for j in range(1, len(vals) - shift):
            r = jnp.where(m[j - 1], vals[j + shift], r)
        return r

    def softplus(v):
        return jnp.maximum(v, 0.0) + jnp.log1p(jnp.exp(-jnp.abs(v)))

    x_k = ladder(kx[:K])
    y_k = ladder(ky[:K])
    w_b = MIN_W + ladder(cwk)
    h_b = MIN_H + ladder(chk)
    d_k = MIN_D + softplus(ladder(u[:K]))
    d_k1 = MIN_D + softplus(ladder(u, shift=1))

    inv_w = 1.0 / w_b
    xi = (x - x_k) * inv_w
    s_k = h_b * inv_w
    omx = 1.0 - xi
    ximx = xi * omx
    xi2 = xi * xi
    num = h_b * (s_k * xi2 + d_k * ximx)
    den = s_k + (d_k1 - s_k + d_k - s_k) * ximx
    r_den = 1.0 / den
    y = y_k + num * r_den
    num_g = (s_k * s_k) * (d_k1 * xi2 + (s_k + s_k) * ximx + d_k * omx * omx)
    dy_dx = num_g * r_den * r_den * inv_w
    log_det = jnp.log(jnp.abs(dy_dx))

    outside = (x <= -TAIL) | (x >= TAIL)
    outr[...] = jnp.where(outside, x, y)
    ldr[...] = jnp.sum(jnp.where(outside, 0.0, log_det), axis=0)


@jax.jit
def _run(x, uw, uh, ud):
    mesh = plsc.VectorSubcoreMesh(core_axis_name="c", subcore_axis_name="s")
    f = pl.kernel(
        _sc_body,
        mesh=mesh,
        compiler_params=pltpu.CompilerParams(needs_layout_passes=False),
        out_type=(
            jax.ShapeDtypeStruct((D, B_SC), jnp.float32),
            jax.ShapeDtypeStruct((B_SC,), jnp.float32),
        ),
        scratch_types=[
            pltpu.VMEM((2 * BW,), jnp.float32),
            pltpu.VMEM((K, 2 * BW), jnp.float32),
            pltpu.VMEM((K, 2 * BW), jnp.float32),
            pltpu.VMEM((K + 1, 2 * BW), jnp.float32),
            pltpu.VMEM((2 * BW,), jnp.float32),
            pltpu.VMEM((BW,), jnp.float32),
            pltpu.SemaphoreType.DMA,
            pltpu.SemaphoreType.DMA,
            pltpu.SemaphoreType.DMA,
            pltpu.SemaphoreType.DMA,
        ],
    )
    x_t = x.T
    uw_t = uw.transpose(1, 2, 0)
    uh_t = uh.transpose(1, 2, 0)
    ud_t = ud.transpose(2, 1, 0)
    out_sc, ld_sc = f(x_t, uw_t, uh_t, ud_t)

    out_tc, ld_tc = pl.pallas_call(
        _tc_body,
        grid=(B_TC // BT,),
        in_specs=[
            pl.BlockSpec((D, BT), lambda i: (0, OFF + i)),
            pl.BlockSpec((D, K, BT), lambda i: (0, 0, OFF + i)),
            pl.BlockSpec((D, K, BT), lambda i: (0, 0, OFF + i)),
            pl.BlockSpec((K + 1, D, BT), lambda i: (0, 0, OFF + i)),
        ],
        out_specs=[
            pl.BlockSpec((D, BT), lambda i: (0, i)),
            pl.BlockSpec((BT,), lambda i: (i,)),
        ],
        out_shape=[
            jax.ShapeDtypeStruct((D, B_TC), jnp.float32),
            jax.ShapeDtypeStruct((B_TC,), jnp.float32),
        ],
    )(x_t, uw_t, uh_t, ud_t)

    out_t = jnp.concatenate([out_sc, out_tc], axis=1)
    ld = jnp.concatenate([ld_sc, ld_tc])
    return out_t.T, ld


def kernel(x, unnormalized_widths, unnormalized_heights, unnormalized_derivatives):
    return _run(x, unnormalized_widths, unnormalized_heights,
                unnormalized_derivatives)
